# Initial kernel scaffold; baseline (speedup 1.0000x reference)
import jax
import jax.numpy as jnp
from jax.experimental import pallas as pl


def kernel(z, W_in, b_in, Wq_in, bq_in, codebooks, Wq_out, bq_out):
    raise NotImplementedError("write your pallas kernel here")


# speedup vs baseline: 2.2834x; 2.2834x over previous
import functools

import jax
import jax.numpy as jnp
from jax.experimental import pallas as pl

N_CB = 8
CB_SIZE = 1024
CB_DIM = 128
ENC = 512
D_IN = 768


def _rvq_body(z_ref, Win_ref, bin_ref, Wqin_ref, bqin_ref, cb_ref,
              Wqout_ref, bqout_ref, codes_ref, loss_ref, *, block_t):
    t = pl.program_id(0)
    f32 = jnp.float32

    zp = jnp.dot(z_ref[...], Win_ref[...], preferred_element_type=f32)
    r = zp + bin_ref[...]
    lsum = jnp.zeros((), dtype=f32)
    for i in range(N_CB):
        e = jnp.dot(r, Wqin_ref[i], preferred_element_type=f32) + bqin_ref[i]
        C = cb_ref[i]
        cn = jnp.sum(C * C, axis=-1)
        ec = jax.lax.dot_general(e, C, (((1,), (1,)), ((), ())),
                                 preferred_element_type=f32)
        dist = (jnp.sum(e * e, axis=-1, keepdims=True) - 2.0 * ec) + cn[None, :]
        idx = jnp.argmin(dist, axis=-1)
        oh = (jax.lax.broadcasted_iota(jnp.int32, (block_t, CB_SIZE), 1)
              == idx[:, None]).astype(f32)
        zq = jnp.dot(oh, C, preferred_element_type=f32)
        lsum = lsum + jnp.sum((e - zq) ** 2)
        out = jnp.dot(zq, Wqout_ref[i], preferred_element_type=f32) + bqout_ref[i]
        r = r - out
        codes_ref[i, :] = idx

    prev = jnp.where(t == 0, jnp.zeros((1, 1), f32), loss_ref[...])
    loss_ref[...] = prev + lsum[None, None]


def kernel(z, W_in, b_in, Wq_in, bq_in, codebooks, Wq_out, bq_out):
    B, T, D = z.shape
    N = B * T
    zf = z.reshape(N, D)
    TB = min(512, N)
    grid = (N // TB,)

    const3 = lambda t: (0, 0, 0)
    const2 = lambda t: (0, 0)
    codes, loss_sum = pl.pallas_call(
        functools.partial(_rvq_body, block_t=TB),
        grid=grid,
        in_specs=[
            pl.BlockSpec((TB, D), lambda t: (t, 0)),
            pl.BlockSpec((D, ENC), const2),
            pl.BlockSpec((1, ENC), const2),
            pl.BlockSpec((N_CB, ENC, CB_DIM), const3),
            pl.BlockSpec((N_CB, 1, CB_DIM), const3),
            pl.BlockSpec((N_CB, CB_SIZE, CB_DIM), const3),
            pl.BlockSpec((N_CB, CB_DIM, ENC), const3),
            pl.BlockSpec((N_CB, 1, ENC), const3),
        ],
        out_specs=[
            pl.BlockSpec((N_CB, TB), lambda t: (0, t)),
            pl.BlockSpec((1, 1), const2),
        ],
        out_shape=[
            jax.ShapeDtypeStruct((N_CB, N), jnp.int32),
            jax.ShapeDtypeStruct((1, 1), jnp.float32),
        ],
    )(zf, W_in, b_in.reshape(1, ENC), Wq_in, bq_in.reshape(N_CB, 1, CB_DIM),
      codebooks, Wq_out, bq_out.reshape(N_CB, 1, ENC))

    codes = codes.T.reshape(B, T, N_CB)
    vq_loss = loss_sum[0, 0] * (1.25 / (N_CB * N * CB_DIM))
    return codes, vq_loss


# speedup vs baseline: 4.2314x; 1.8531x over previous
import functools

import jax
import jax.numpy as jnp
from jax.experimental import pallas as pl

N_CB = 8
CB_SIZE = 1024
CB_DIM = 128
ENC = 512
D_IN = 768


def _rvq_body(z_ref, Win_ref, bin_ref, Wqin_ref, bqin_ref, cb_ref, ct2_ref,
              cn_ref, Wqout_ref, bqout_ref, codes_ref, loss_ref,
              *, block_t, n_half):
    t = pl.program_id(0)
    f32 = jnp.float32
    h = block_t // n_half

    zp = jnp.dot(z_ref[...], Win_ref[...], preferred_element_type=f32)
    zp = zp + bin_ref[...]
    iota = jax.lax.broadcasted_iota(jnp.int32, (h, CB_SIZE), 1)
    rs = [zp[k * h:(k + 1) * h] for k in range(n_half)]
    accs = [jnp.zeros((h, CB_DIM), f32) for _ in range(n_half)]
    ks = range(n_half)
    for i in range(N_CB):
        C = cb_ref[i]
        es = [jnp.dot(rs[k], Wqin_ref[i], preferred_element_type=f32)
              + bqin_ref[i] for k in ks]
        scores = [jnp.dot(es[k], ct2_ref[i], preferred_element_type=f32)
                  + cn_ref[i] for k in ks]
        idxs = [jnp.argmin(scores[k], axis=-1) for k in ks]
        ohs = [(iota == idxs[k][:, None]).astype(f32) for k in ks]
        zqs = [jnp.dot(ohs[k], C, preferred_element_type=f32) for k in ks]
        accs = [accs[k] + (es[k] - zqs[k]) ** 2 for k in ks]
        outs = [jnp.dot(zqs[k], Wqout_ref[i], preferred_element_type=f32)
                for k in ks]
        rs = [rs[k] - (outs[k] + bqout_ref[i]) for k in ks]
        for k in ks:
            codes_ref[i, k * h:(k + 1) * h] = idxs[k]

    lsum = sum(jnp.sum(a) for a in accs)
    prev = jnp.where(t == 0, jnp.zeros((1, 1), f32), loss_ref[...])
    loss_ref[...] = prev + lsum[None, None]


def kernel(z, W_in, b_in, Wq_in, bq_in, codebooks, Wq_out, bq_out):
    B, T, D = z.shape
    N = B * T
    zf = z.reshape(N, D)
    TB = min(1024, N)
    NH = 8 if TB >= 1024 else 1
    grid = (N // TB,)

    ct2 = -2.0 * codebooks.transpose(0, 2, 1)
    cn = jnp.sum(codebooks * codebooks, axis=-1)[:, None, :]

    const3 = lambda t: (0, 0, 0)
    const2 = lambda t: (0, 0)
    codes, loss_sum = pl.pallas_call(
        functools.partial(_rvq_body, block_t=TB, n_half=NH),
        grid=grid,
        in_specs=[
            pl.BlockSpec((TB, D), lambda t: (t, 0)),
            pl.BlockSpec((D, ENC), const2),
            pl.BlockSpec((1, ENC), const2),
            pl.BlockSpec((N_CB, ENC, CB_DIM), const3),
            pl.BlockSpec((N_CB, 1, CB_DIM), const3),
            pl.BlockSpec((N_CB, CB_SIZE, CB_DIM), const3),
            pl.BlockSpec((N_CB, CB_DIM, CB_SIZE), const3),
            pl.BlockSpec((N_CB, 1, CB_SIZE), const3),
            pl.BlockSpec((N_CB, CB_DIM, ENC), const3),
            pl.BlockSpec((N_CB, 1, ENC), const3),
        ],
        out_specs=[
            pl.BlockSpec((N_CB, TB), lambda t: (0, t)),
            pl.BlockSpec((1, 1), const2),
        ],
        out_shape=[
            jax.ShapeDtypeStruct((N_CB, N), jnp.int32),
            jax.ShapeDtypeStruct((1, 1), jnp.float32),
        ],
    )(zf, W_in, b_in.reshape(1, ENC), Wq_in, bq_in.reshape(N_CB, 1, CB_DIM),
      codebooks, ct2, cn, Wq_out, bq_out.reshape(N_CB, 1, ENC))

    codes = codes.T.reshape(B, T, N_CB)
    vq_loss = loss_sum[0, 0] * (1.25 / (N_CB * N * CB_DIM))
    return codes, vq_loss
